# baseline (speedup 1.0000x reference)
import functools

import jax
import jax.numpy as jnp
from jax import lax
from jax.experimental import pallas as pl
from jax.experimental.pallas import tpu as pltpu
from jax.experimental.pallas import tpu_sc as plsc

N = 10000
E = 320000
D = 128
LANES = 16
NC = 2
NS = 16
NW = NC * NS
EB = 128
BPW = 80
NBLK = NW * BPW
EPAD = NBLK * EB
NPAD = N + 64
NPAD2 = 10240
RPS = 632
LAST_R = NPAD - 15 * RPS

_mesh = plsc.VectorSubcoreMesh(core_axis_name="c", subcore_axis_name="s")


def _zero_vmem(ref, nrows, ncols):
    zero16 = jnp.zeros((LANES,), jnp.float32)

    def zrow(i, c):
        for c8 in range(ncols // LANES):
            ref[i, pl.ds(c8 * LANES, LANES)] = zero16
        return c
    lax.fori_loop(0, nrows, zrow, 0)


def _zero_shared(dst_sh, src_buf, r0, nrows):
    nb = src_buf.shape[0]
    for k in range(nrows // nb):
        pltpu.sync_copy(src_buf, dst_sh.at[pl.ds(r0 + k * nb, nb)])
    tail = nrows % nb
    if tail:
        pltpu.sync_copy(src_buf.at[pl.ds(0, tail)],
                        dst_sh.at[pl.ds(r0 + (nrows // nb) * nb, tail)])


EBA = 64
NBLKA = EPAD // EBA
BPWA = NBLKA // NW
IDXCA = 32
NCH = BPWA // IDXCA

_agg_scratch = [
    pltpu.VMEM_SHARED((NPAD, D), jnp.float32),
    pltpu.VMEM((IDXCA, EBA), jnp.int32),
    pltpu.VMEM((IDXCA, EBA), jnp.int32),
    pltpu.VMEM((IDXCA, EBA), jnp.int32),
    pltpu.VMEM((IDXCA, EBA), jnp.int32),
    pltpu.VMEM((EBA, D), jnp.float32),
    pltpu.VMEM((EBA, D), jnp.float32),
    pltpu.SemaphoreType.DMA,
    pltpu.SemaphoreType.DMA,
    pltpu.SemaphoreType.DMA,
    pltpu.SemaphoreType.DMA,
]
_agg_out_type = [jax.ShapeDtypeStruct((NC, NPAD, D), jnp.float32)]


def _sc_agg_body(feat, srcb, dstb, agg_out, agg_sh,
                 srcA, dstA, srcB, dstB, rows_a, rows_b,
                 sga, sgb, ssa, ssb):
    cid = lax.axis_index("c")
    sid = lax.axis_index("s")
    w = cid * NS + sid
    r0 = sid * RPS

    _zero_vmem(rows_a, EBA, D)
    _zero_vmem(rows_b, EBA, D)

    @pl.when(sid < NS - 1)
    def _():
        _zero_shared(agg_sh, rows_a, r0, RPS)

    @pl.when(sid == NS - 1)
    def _():
        _zero_shared(agg_sh, rows_a, r0, LAST_R)

    plsc.subcore_barrier()

    base = w * BPWA
    banks = ((srcA, dstA), (srcB, dstB))
    bufs = ((rows_a, sga, ssa), (rows_b, sgb, ssb))

    def load_chunk(c, bank):
        pltpu.sync_copy(srcb.at[pl.ds(base + c * IDXCA, IDXCA)], bank[0])
        pltpu.sync_copy(dstb.at[pl.ds(base + c * IDXCA, IDXCA)], bank[1])

    load_chunk(0, banks[0])
    pltpu.async_copy(feat.at[banks[0][0].at[0]], rows_a, sga)
    pltpu.async_copy(rows_b, agg_sh.at[banks[0][1].at[0]], ssb, add=True)

    for c in range(NCH):
        cur = banks[c % 2]
        nxt = banks[(c + 1) % 2]
        if c + 1 < NCH:
            load_chunk(c + 1, nxt)
        last_chunk = c + 1 == NCH

        def pair(m2, carry):
            for t in (0, 1):
                k = 2 * m2 + t
                buf, sg, ss = bufs[t]
                obuf, osg, oss = bufs[1 - t]
                pltpu.make_async_copy(feat.at[cur[0].at[k]], buf, sg).wait()
                pltpu.make_async_copy(obuf, agg_sh.at[cur[1].at[k]],
                                      oss).wait()
                if t == 0:
                    pltpu.async_copy(feat.at[cur[0].at[k + 1]], obuf, osg)
                else:
                    @pl.when(m2 < IDXCA // 2 - 1)
                    def _():
                        pltpu.async_copy(feat.at[cur[0].at[k + 1]], obuf, osg)
                    if not last_chunk:
                        @pl.when(m2 == IDXCA // 2 - 1)
                        def _():
                            pltpu.async_copy(feat.at[nxt[0].at[0]], obuf, osg)
                pltpu.async_copy(buf, agg_sh.at[cur[1].at[k]], ss, add=True)
            return carry
        lax.fori_loop(0, IDXCA // 2, pair, 0)

    pltpu.make_async_copy(rows_b, agg_sh.at[banks[(NCH - 1) % 2][1].at[IDXCA - 1]],
                          ssb).wait()

    plsc.subcore_barrier()

    @pl.when(sid < NS - 1)
    def _():
        pltpu.sync_copy(agg_sh.at[pl.ds(r0, RPS)],
                        agg_out.at[cid, pl.ds(r0, RPS)])

    @pl.when(sid == NS - 1)
    def _():
        pltpu.sync_copy(agg_sh.at[pl.ds(r0, LAST_R)],
                        agg_out.at[cid, pl.ds(r0, LAST_R)])


_sc_agg = pl.kernel(_sc_agg_body, mesh=_mesh, out_type=_agg_out_type,
                    scratch_types=_agg_scratch)


_cnt_out_type = [jax.ShapeDtypeStruct((NW, 1, NPAD2), jnp.float32)]
_cnt_scratch = [
    pltpu.VMEM((1, NPAD2), jnp.float32),
    pltpu.VMEM((BPW, EB), jnp.int32),
]


def _sc_cnt_body(dstb, cnt_out, hist, dst_v):
    cid = lax.axis_index("c")
    sid = lax.axis_index("s")
    w = cid * NS + sid

    zero16 = jnp.zeros((LANES,), jnp.float32)

    def zh(i, c):
        hist[0, pl.ds(i * LANES, LANES)] = zero16
        return c
    lax.fori_loop(0, NPAD2 // LANES, zh, 0)

    pltpu.sync_copy(dstb.at[pl.ds(w * BPW, BPW)], dst_v)

    zero16i = jnp.zeros((LANES,), jnp.int32)
    one16 = jnp.ones((LANES,), jnp.float32)

    def body(j, c):
        for k in range(EB // LANES):
            idx16 = dst_v[j, pl.ds(k * LANES, LANES)]
            plsc.addupdate_scatter(hist, [zero16i, idx16], one16)
        return c
    lax.fori_loop(0, BPW, body, 0)

    pltpu.sync_copy(hist, cnt_out.at[w])


_sc_cnt = pl.kernel(
    _sc_cnt_body, mesh=_mesh, out_type=_cnt_out_type,
    scratch_types=_cnt_scratch,
    compiler_params=pltpu.CompilerParams(needs_layout_passes=False))

BN = 2048


def _dense_body(relu):
    def body(aggp, cntp, x, wl, bl, wr, o):
        i = pl.program_id(0)
        cp = cntp[:, 0, pl.ds(i * BN, BN)]
        cnt = jnp.maximum(jnp.sum(cp, axis=0), 1.0)[:, None]
        a = aggp[...]
        mean = (a[0] + a[1]) / cnt
        r = (jnp.dot(mean, wl[...], preferred_element_type=jnp.float32)
             + jnp.dot(x[...], wr[...], preferred_element_type=jnp.float32)
             + bl[...])
        o[...] = jnp.maximum(r, 0.0) if relu else r
    return body


def _dense(aggp, cntp, x, Wl, bl, Wr, relu):
    return pl.pallas_call(
        _dense_body(relu),
        grid=(pl.cdiv(N, BN),),
        in_specs=[
            pl.BlockSpec((2, BN, D), lambda i: (0, i, 0)),
            pl.BlockSpec((NW, 1, NPAD2), lambda i: (0, 0, 0)),
            pl.BlockSpec((BN, D), lambda i: (i, 0)),
            pl.BlockSpec((D, D), lambda i: (0, 0)),
            pl.BlockSpec((1, D), lambda i: (0, 0)),
            pl.BlockSpec((D, D), lambda i: (0, 0)),
        ],
        out_specs=pl.BlockSpec((BN, D), lambda i: (i, 0)),
        out_shape=jax.ShapeDtypeStruct((N, D), jnp.float32),
    )(aggp, cntp, x, Wl, bl.reshape(1, D), Wr)


def kernel(x, edge_index, W1l, b1l, W1r, W2l, b2l, W2r):
    npad = EPAD - E
    fill = jnp.arange(npad, dtype=jnp.int32) % 64
    src = jnp.concatenate([edge_index[0], fill])
    dst = jnp.concatenate([edge_index[1], N + fill])
    srcb = src.reshape(NBLKA, EBA)
    dstb = dst.reshape(NBLKA, EBA)
    (cnt,) = _sc_cnt(dst.reshape(NBLK, EB))
    (agg1,) = _sc_agg(x, srcb, dstb)
    h = _dense(agg1, cnt, x, W1l, b1l, W1r, relu=True)
    (agg2,) = _sc_agg(h, srcb, dstb)
    out = _dense(agg2, cnt, h, W2l, b2l, W2r, relu=False)
    return out


# speedup vs baseline: 1.2828x; 1.2828x over previous
import functools

import jax
import jax.numpy as jnp
from jax import lax
from jax.experimental import pallas as pl
from jax.experimental.pallas import tpu as pltpu
from jax.experimental.pallas import tpu_sc as plsc

N = 10000
E = 320000
D = 128
LANES = 16
NC = 2
NS = 16
NW = NC * NS
EB = 128
BPW = 80
NBLK = NW * BPW
EPAD = NBLK * EB
NPAD = N + 64
NPAD2 = 10240
RPS = 632
LAST_R = NPAD - 15 * RPS

_mesh = plsc.VectorSubcoreMesh(core_axis_name="c", subcore_axis_name="s")


def _zero_vmem(ref, nrows, ncols):
    zero16 = jnp.zeros((LANES,), jnp.float32)

    def zrow(i, c):
        for c8 in range(ncols // LANES):
            ref[i, pl.ds(c8 * LANES, LANES)] = zero16
        return c
    lax.fori_loop(0, nrows, zrow, 0)


def _zero_shared(dst_sh, src_buf, r0, nrows):
    nb = src_buf.shape[0]
    for k in range(nrows // nb):
        pltpu.sync_copy(src_buf, dst_sh.at[pl.ds(r0 + k * nb, nb)])
    tail = nrows % nb
    if tail:
        pltpu.sync_copy(src_buf.at[pl.ds(0, tail)],
                        dst_sh.at[pl.ds(r0 + (nrows // nb) * nb, tail)])


EBA = 64
NBLKA = EPAD // EBA
BPWA = NBLKA // NW
IDXCA = 32
NCH = BPWA // IDXCA

_agg_scratch = [
    pltpu.VMEM_SHARED((NPAD, D), jnp.float32),
    pltpu.VMEM((IDXCA, EBA), jnp.int32),
    pltpu.VMEM((IDXCA, EBA), jnp.int32),
    pltpu.VMEM((IDXCA, EBA), jnp.int32),
    pltpu.VMEM((IDXCA, EBA), jnp.int32),
    pltpu.VMEM((EBA, D), jnp.float32),
    pltpu.VMEM((EBA, D), jnp.float32),
    pltpu.SemaphoreType.DMA,
    pltpu.SemaphoreType.DMA,
]
_agg_out_type = [jax.ShapeDtypeStruct((NC, NPAD, D), jnp.float32)]


def _sc_agg_body(feat, srcb, dstb, agg_out, agg_sh,
                 srcA, dstA, srcB, dstB, rows_a, rows_b, sga, sgb):
    cid = lax.axis_index("c")
    sid = lax.axis_index("s")
    w = cid * NS + sid
    r0 = sid * RPS

    _zero_vmem(rows_a, EBA, D)

    @pl.when(sid < NS - 1)
    def _():
        _zero_shared(agg_sh, rows_a, r0, RPS)

    @pl.when(sid == NS - 1)
    def _():
        _zero_shared(agg_sh, rows_a, r0, LAST_R)

    plsc.subcore_barrier()

    base = w * BPWA
    banks = ((srcA, dstA), (srcB, dstB))
    bufs = ((rows_a, sga), (rows_b, sgb))

    def load_chunk(c, bank):
        pltpu.sync_copy(srcb.at[pl.ds(base + c * IDXCA, IDXCA)], bank[0])
        pltpu.sync_copy(dstb.at[pl.ds(base + c * IDXCA, IDXCA)], bank[1])

    load_chunk(0, banks[0])
    pltpu.async_copy(feat.at[banks[0][0].at[0]], rows_a, sga)
    pltpu.async_copy(feat.at[banks[0][0].at[1]], rows_b, sgb)

    for c in range(NCH):
        cur = banks[c % 2]
        nxt = banks[(c + 1) % 2]
        if c + 1 < NCH:
            load_chunk(c + 1, nxt)

        def pair(m2, carry):
            for t, (buf, sem) in enumerate(bufs):
                k = 2 * m2 + t
                pltpu.make_async_copy(feat.at[cur[0].at[k]], buf, sem).wait()
                pltpu.sync_copy(buf, agg_sh.at[cur[1].at[k]], add=True)

                @pl.when(m2 < IDXCA // 2 - 1)
                def _():
                    pltpu.async_copy(feat.at[cur[0].at[k + 2]], buf, sem)
                if c + 1 < NCH:
                    @pl.when(m2 == IDXCA // 2 - 1)
                    def _():
                        pltpu.async_copy(feat.at[nxt[0].at[t]], buf, sem)
            return carry
        lax.fori_loop(0, IDXCA // 2, pair, 0)

    plsc.subcore_barrier()

    @pl.when(sid < NS - 1)
    def _():
        pltpu.sync_copy(agg_sh.at[pl.ds(r0, RPS)],
                        agg_out.at[cid, pl.ds(r0, RPS)])

    @pl.when(sid == NS - 1)
    def _():
        pltpu.sync_copy(agg_sh.at[pl.ds(r0, LAST_R)],
                        agg_out.at[cid, pl.ds(r0, LAST_R)])


_sc_agg = pl.kernel(_sc_agg_body, mesh=_mesh, out_type=_agg_out_type,
                    scratch_types=_agg_scratch)


_cnt_out_type = [jax.ShapeDtypeStruct((NW, 1, NPAD2), jnp.float32)]
_cnt_scratch = [
    pltpu.VMEM((1, NPAD2), jnp.float32),
    pltpu.VMEM((BPW, EB), jnp.int32),
]


def _sc_cnt_body(dstb, cnt_out, hist, dst_v):
    cid = lax.axis_index("c")
    sid = lax.axis_index("s")
    w = cid * NS + sid

    zero16 = jnp.zeros((LANES,), jnp.float32)

    def zh(i, c):
        hist[0, pl.ds(i * LANES, LANES)] = zero16
        return c
    lax.fori_loop(0, NPAD2 // LANES, zh, 0)

    pltpu.sync_copy(dstb.at[pl.ds(w * BPW, BPW)], dst_v)

    zero16i = jnp.zeros((LANES,), jnp.int32)
    one16 = jnp.ones((LANES,), jnp.float32)

    def body(j, c):
        for k in range(EB // LANES):
            idx16 = dst_v[j, pl.ds(k * LANES, LANES)]
            plsc.addupdate_scatter(hist, [zero16i, idx16], one16)
        return c
    lax.fori_loop(0, BPW, body, 0)

    pltpu.sync_copy(hist, cnt_out.at[w])


_sc_cnt = pl.kernel(
    _sc_cnt_body, mesh=_mesh, out_type=_cnt_out_type,
    scratch_types=_cnt_scratch,
    compiler_params=pltpu.CompilerParams(needs_layout_passes=False))

BN = 2048


def _dense_body(relu):
    def body(aggp, cntp, x, wl, bl, wr, o):
        i = pl.program_id(0)
        cp = cntp[:, 0, pl.ds(i * BN, BN)]
        cnt = jnp.maximum(jnp.sum(cp, axis=0), 1.0)[:, None]
        a = aggp[...]
        mean = (a[0] + a[1]) / cnt
        r = (jnp.dot(mean, wl[...], preferred_element_type=jnp.float32)
             + jnp.dot(x[...], wr[...], preferred_element_type=jnp.float32)
             + bl[...])
        o[...] = jnp.maximum(r, 0.0) if relu else r
    return body


def _dense(aggp, cntp, x, Wl, bl, Wr, relu):
    return pl.pallas_call(
        _dense_body(relu),
        grid=(pl.cdiv(N, BN),),
        in_specs=[
            pl.BlockSpec((2, BN, D), lambda i: (0, i, 0)),
            pl.BlockSpec((NW, 1, NPAD2), lambda i: (0, 0, 0)),
            pl.BlockSpec((BN, D), lambda i: (i, 0)),
            pl.BlockSpec((D, D), lambda i: (0, 0)),
            pl.BlockSpec((1, D), lambda i: (0, 0)),
            pl.BlockSpec((D, D), lambda i: (0, 0)),
        ],
        out_specs=pl.BlockSpec((BN, D), lambda i: (i, 0)),
        out_shape=jax.ShapeDtypeStruct((N, D), jnp.float32),
    )(aggp, cntp, x, Wl, bl.reshape(1, D), Wr)


def kernel(x, edge_index, W1l, b1l, W1r, W2l, b2l, W2r):
    npad = EPAD - E
    fill = jnp.arange(npad, dtype=jnp.int32) % 64
    src = jnp.concatenate([edge_index[0], fill])
    dst = jnp.concatenate([edge_index[1], N + fill])
    srcb = src.reshape(NBLKA, EBA)
    dstb = dst.reshape(NBLKA, EBA)
    (cnt,) = _sc_cnt(dst.reshape(NBLK, EB))
    (agg1,) = _sc_agg(x, srcb, dstb)
    h = _dense(agg1, cnt, x, W1l, b1l, W1r, relu=True)
    (agg2,) = _sc_agg(h, srcb, dstb)
    out = _dense(agg2, cnt, h, W2l, b2l, W2r, relu=False)
    return out


# speedup vs baseline: 1.3800x; 1.0757x over previous
import functools

import jax
import jax.numpy as jnp
from jax import lax
from jax.experimental import pallas as pl
from jax.experimental.pallas import tpu as pltpu
from jax.experimental.pallas import tpu_sc as plsc

N = 10000
E = 320000
D = 128
LANES = 16
NC = 2
NS = 16
NW = NC * NS
EB = 128
BPW = 80
NBLK = NW * BPW
EPAD = NBLK * EB
NPAD = N + 64
NPAD2 = 10240
RPS = 632
LAST_R = NPAD - 15 * RPS

_mesh = plsc.VectorSubcoreMesh(core_axis_name="c", subcore_axis_name="s")


def _zero_vmem(ref, nrows, ncols):
    zero16 = jnp.zeros((LANES,), jnp.float32)

    def zrow(i, c):
        for c8 in range(ncols // LANES):
            ref[i, pl.ds(c8 * LANES, LANES)] = zero16
        return c
    lax.fori_loop(0, nrows, zrow, 0)


def _zero_shared(dst_sh, src_buf, r0, nrows):
    nb = src_buf.shape[0]
    for k in range(nrows // nb):
        pltpu.sync_copy(src_buf, dst_sh.at[pl.ds(r0 + k * nb, nb)])
    tail = nrows % nb
    if tail:
        pltpu.sync_copy(src_buf.at[pl.ds(0, tail)],
                        dst_sh.at[pl.ds(r0 + (nrows // nb) * nb, tail)])


EBA = 80
NBLKA = EPAD // EBA
BPWA = NBLKA // NW
IDXCA = 32
NCH = BPWA // IDXCA

_agg_scratch = [
    pltpu.VMEM_SHARED((NPAD, D), jnp.float32),
    pltpu.VMEM((IDXCA, EBA), jnp.int32),
    pltpu.VMEM((IDXCA, EBA), jnp.int32),
    pltpu.VMEM((IDXCA, EBA), jnp.int32),
    pltpu.VMEM((IDXCA, EBA), jnp.int32),
    pltpu.VMEM((EBA, D), jnp.float32),
    pltpu.VMEM((EBA, D), jnp.float32),
    pltpu.SemaphoreType.DMA,
    pltpu.SemaphoreType.DMA,
]
_agg_out_type = [jax.ShapeDtypeStruct((NC, NPAD, D), jnp.float32)]


def _sc_agg_body(feat, srcb, dstb, agg_out, agg_sh,
                 srcA, dstA, srcB, dstB, rows_a, rows_b, sga, sgb):
    cid = lax.axis_index("c")
    sid = lax.axis_index("s")
    w = cid * NS + sid
    r0 = sid * RPS

    _zero_vmem(rows_a, EBA, D)

    @pl.when(sid < NS - 1)
    def _():
        _zero_shared(agg_sh, rows_a, r0, RPS)

    @pl.when(sid == NS - 1)
    def _():
        _zero_shared(agg_sh, rows_a, r0, LAST_R)

    plsc.subcore_barrier()

    base = w * BPWA
    banks = ((srcA, dstA), (srcB, dstB))
    bufs = ((rows_a, sga), (rows_b, sgb))

    def load_chunk(c, bank):
        pltpu.sync_copy(srcb.at[pl.ds(base + c * IDXCA, IDXCA)], bank[0])
        pltpu.sync_copy(dstb.at[pl.ds(base + c * IDXCA, IDXCA)], bank[1])

    load_chunk(0, banks[0])
    pltpu.async_copy(feat.at[banks[0][0].at[0]], rows_a, sga)
    pltpu.async_copy(feat.at[banks[0][0].at[1]], rows_b, sgb)

    for c in range(NCH):
        cur = banks[c % 2]
        nxt = banks[(c + 1) % 2]
        if c + 1 < NCH:
            load_chunk(c + 1, nxt)

        def pair(m2, carry):
            for t, (buf, sem) in enumerate(bufs):
                k = 2 * m2 + t
                pltpu.make_async_copy(feat.at[cur[0].at[k]], buf, sem).wait()
                pltpu.sync_copy(buf, agg_sh.at[cur[1].at[k]], add=True)

                @pl.when(m2 < IDXCA // 2 - 1)
                def _():
                    pltpu.async_copy(feat.at[cur[0].at[k + 2]], buf, sem)
                if c + 1 < NCH:
                    @pl.when(m2 == IDXCA // 2 - 1)
                    def _():
                        pltpu.async_copy(feat.at[nxt[0].at[t]], buf, sem)
            return carry
        lax.fori_loop(0, IDXCA // 2, pair, 0)

    plsc.subcore_barrier()

    @pl.when(sid < NS - 1)
    def _():
        pltpu.sync_copy(agg_sh.at[pl.ds(r0, RPS)],
                        agg_out.at[cid, pl.ds(r0, RPS)])

    @pl.when(sid == NS - 1)
    def _():
        pltpu.sync_copy(agg_sh.at[pl.ds(r0, LAST_R)],
                        agg_out.at[cid, pl.ds(r0, LAST_R)])


_sc_agg = pl.kernel(_sc_agg_body, mesh=_mesh, out_type=_agg_out_type,
                    scratch_types=_agg_scratch)


_cnt_out_type = [jax.ShapeDtypeStruct((NW, 1, NPAD2), jnp.float32)]
_cnt_scratch = [
    pltpu.VMEM((1, NPAD2), jnp.float32),
    pltpu.VMEM((BPW, EB), jnp.int32),
]


def _sc_cnt_body(dstb, cnt_out, hist, dst_v):
    cid = lax.axis_index("c")
    sid = lax.axis_index("s")
    w = cid * NS + sid

    zero16 = jnp.zeros((LANES,), jnp.float32)

    def zh(i, c):
        hist[0, pl.ds(i * LANES, LANES)] = zero16
        return c
    lax.fori_loop(0, NPAD2 // LANES, zh, 0)

    pltpu.sync_copy(dstb.at[pl.ds(w * BPW, BPW)], dst_v)

    zero16i = jnp.zeros((LANES,), jnp.int32)
    one16 = jnp.ones((LANES,), jnp.float32)

    def body(j, c):
        for k in range(EB // LANES):
            idx16 = dst_v[j, pl.ds(k * LANES, LANES)]
            plsc.addupdate_scatter(hist, [zero16i, idx16], one16)
        return c
    lax.fori_loop(0, BPW, body, 0)

    pltpu.sync_copy(hist, cnt_out.at[w])


_sc_cnt = pl.kernel(
    _sc_cnt_body, mesh=_mesh, out_type=_cnt_out_type,
    scratch_types=_cnt_scratch,
    compiler_params=pltpu.CompilerParams(needs_layout_passes=False))

BN = 2048


def _dense_body(relu):
    def body(aggp, cntp, x, wl, bl, wr, o):
        i = pl.program_id(0)
        cp = cntp[:, 0, pl.ds(i * BN, BN)]
        cnt = jnp.maximum(jnp.sum(cp, axis=0), 1.0)[:, None]
        a = aggp[...]
        mean = (a[0] + a[1]) / cnt
        r = (jnp.dot(mean, wl[...], preferred_element_type=jnp.float32)
             + jnp.dot(x[...], wr[...], preferred_element_type=jnp.float32)
             + bl[...])
        o[...] = jnp.maximum(r, 0.0) if relu else r
    return body


def _dense(aggp, cntp, x, Wl, bl, Wr, relu):
    return pl.pallas_call(
        _dense_body(relu),
        grid=(pl.cdiv(N, BN),),
        in_specs=[
            pl.BlockSpec((2, BN, D), lambda i: (0, i, 0)),
            pl.BlockSpec((NW, 1, NPAD2), lambda i: (0, 0, 0)),
            pl.BlockSpec((BN, D), lambda i: (i, 0)),
            pl.BlockSpec((D, D), lambda i: (0, 0)),
            pl.BlockSpec((1, D), lambda i: (0, 0)),
            pl.BlockSpec((D, D), lambda i: (0, 0)),
        ],
        out_specs=pl.BlockSpec((BN, D), lambda i: (i, 0)),
        out_shape=jax.ShapeDtypeStruct((N, D), jnp.float32),
    )(aggp, cntp, x, Wl, bl.reshape(1, D), Wr)


def kernel(x, edge_index, W1l, b1l, W1r, W2l, b2l, W2r):
    npad = EPAD - E
    fill = jnp.arange(npad, dtype=jnp.int32) % 64
    src = jnp.concatenate([edge_index[0], fill])
    dst = jnp.concatenate([edge_index[1], N + fill])
    srcb = src.reshape(NBLKA, EBA)
    dstb = dst.reshape(NBLKA, EBA)
    (cnt,) = _sc_cnt(dst.reshape(NBLK, EB))
    (agg1,) = _sc_agg(x, srcb, dstb)
    h = _dense(agg1, cnt, x, W1l, b1l, W1r, relu=True)
    (agg2,) = _sc_agg(h, srcb, dstb)
    out = _dense(agg2, cnt, h, W2l, b2l, W2r, relu=False)
    return out


# speedup vs baseline: 1.4416x; 1.0446x over previous
import functools

import jax
import jax.numpy as jnp
from jax import lax
from jax.experimental import pallas as pl
from jax.experimental.pallas import tpu as pltpu
from jax.experimental.pallas import tpu_sc as plsc

N = 10000
E = 320000
D = 128
LANES = 16
NC = 2
NS = 16
NW = NC * NS
EB = 128
BPW = 80
NBLK = NW * BPW
EPAD = NBLK * EB
NPAD = N + 16
NPAD2 = 10240
RPS = 632
LAST_R = NPAD - 15 * RPS

_mesh = plsc.VectorSubcoreMesh(core_axis_name="c", subcore_axis_name="s")


def _zero_vmem(ref, nrows, ncols):
    zero16 = jnp.zeros((LANES,), jnp.float32)

    def zrow(i, c):
        for c8 in range(ncols // LANES):
            ref[i, pl.ds(c8 * LANES, LANES)] = zero16
        return c
    lax.fori_loop(0, nrows, zrow, 0)


def _zero_shared(dst_sh, src_buf, r0, nrows):
    nb = src_buf.shape[0]
    for k in range(nrows // nb):
        pltpu.sync_copy(src_buf, dst_sh.at[pl.ds(r0 + k * nb, nb)])
    tail = nrows % nb
    if tail:
        pltpu.sync_copy(src_buf.at[pl.ds(0, tail)],
                        dst_sh.at[pl.ds(r0 + (nrows // nb) * nb, tail)])


EBA = 128
NBLKA = EPAD // EBA
BPWA = NBLKA // NW
IDXCA = 8
NCH = BPWA // IDXCA

_agg_scratch = [
    pltpu.VMEM_SHARED((NPAD, D), jnp.float32),
    pltpu.VMEM((IDXCA, EBA), jnp.int32),
    pltpu.VMEM((IDXCA, EBA), jnp.int32),
    pltpu.VMEM((IDXCA, EBA), jnp.int32),
    pltpu.VMEM((IDXCA, EBA), jnp.int32),
    pltpu.VMEM((EBA, D), jnp.float32),
    pltpu.VMEM((EBA, D), jnp.float32),
    pltpu.SemaphoreType.DMA,
    pltpu.SemaphoreType.DMA,
]
_agg_out_type = [jax.ShapeDtypeStruct((NC, NPAD, D), jnp.float32)]


def _sc_agg_body(feat, srcb, dstb, agg_out, agg_sh,
                 srcA, dstA, srcB, dstB, rows_a, rows_b, sga, sgb):
    cid = lax.axis_index("c")
    sid = lax.axis_index("s")
    w = cid * NS + sid
    r0 = sid * RPS

    _zero_vmem(rows_a, EBA, D)

    @pl.when(sid < NS - 1)
    def _():
        _zero_shared(agg_sh, rows_a, r0, RPS)

    @pl.when(sid == NS - 1)
    def _():
        _zero_shared(agg_sh, rows_a, r0, LAST_R)

    plsc.subcore_barrier()

    base = w * BPWA
    banks = ((srcA, dstA), (srcB, dstB))
    bufs = ((rows_a, sga), (rows_b, sgb))

    def load_chunk(c, bank):
        pltpu.sync_copy(srcb.at[pl.ds(base + c * IDXCA, IDXCA)], bank[0])
        pltpu.sync_copy(dstb.at[pl.ds(base + c * IDXCA, IDXCA)], bank[1])

    load_chunk(0, banks[0])
    pltpu.async_copy(feat.at[banks[0][0].at[0]], rows_a, sga)
    pltpu.async_copy(feat.at[banks[0][0].at[1]], rows_b, sgb)

    for c in range(NCH):
        cur = banks[c % 2]
        nxt = banks[(c + 1) % 2]
        if c + 1 < NCH:
            load_chunk(c + 1, nxt)

        def pair(m2, carry):
            for t, (buf, sem) in enumerate(bufs):
                k = 2 * m2 + t
                pltpu.make_async_copy(feat.at[cur[0].at[k]], buf, sem).wait()
                pltpu.sync_copy(buf, agg_sh.at[cur[1].at[k]], add=True)

                @pl.when(m2 < IDXCA // 2 - 1)
                def _():
                    pltpu.async_copy(feat.at[cur[0].at[k + 2]], buf, sem)
                if c + 1 < NCH:
                    @pl.when(m2 == IDXCA // 2 - 1)
                    def _():
                        pltpu.async_copy(feat.at[nxt[0].at[t]], buf, sem)
            return carry
        lax.fori_loop(0, IDXCA // 2, pair, 0)

    plsc.subcore_barrier()

    @pl.when(sid < NS - 1)
    def _():
        pltpu.sync_copy(agg_sh.at[pl.ds(r0, RPS)],
                        agg_out.at[cid, pl.ds(r0, RPS)])

    @pl.when(sid == NS - 1)
    def _():
        pltpu.sync_copy(agg_sh.at[pl.ds(r0, LAST_R)],
                        agg_out.at[cid, pl.ds(r0, LAST_R)])


_sc_agg = pl.kernel(_sc_agg_body, mesh=_mesh, out_type=_agg_out_type,
                    scratch_types=_agg_scratch)


_cnt_out_type = [jax.ShapeDtypeStruct((NW, 1, NPAD2), jnp.float32)]
_cnt_scratch = [
    pltpu.VMEM((1, NPAD2), jnp.float32),
    pltpu.VMEM((BPW, EB), jnp.int32),
]


def _sc_cnt_body(dstb, cnt_out, hist, dst_v):
    cid = lax.axis_index("c")
    sid = lax.axis_index("s")
    w = cid * NS + sid

    zero16 = jnp.zeros((LANES,), jnp.float32)

    def zh(i, c):
        hist[0, pl.ds(i * LANES, LANES)] = zero16
        return c
    lax.fori_loop(0, NPAD2 // LANES, zh, 0)

    pltpu.sync_copy(dstb.at[pl.ds(w * BPW, BPW)], dst_v)

    zero16i = jnp.zeros((LANES,), jnp.int32)
    one16 = jnp.ones((LANES,), jnp.float32)

    def body(j, c):
        for k in range(EB // LANES):
            idx16 = dst_v[j, pl.ds(k * LANES, LANES)]
            plsc.addupdate_scatter(hist, [zero16i, idx16], one16)
        return c
    lax.fori_loop(0, BPW, body, 0)

    pltpu.sync_copy(hist, cnt_out.at[w])


_sc_cnt = pl.kernel(
    _sc_cnt_body, mesh=_mesh, out_type=_cnt_out_type,
    scratch_types=_cnt_scratch,
    compiler_params=pltpu.CompilerParams(needs_layout_passes=False))

BN = 2048


def _dense_body(relu):
    def body(aggp, cntp, x, wl, bl, wr, o):
        i = pl.program_id(0)
        cp = cntp[:, 0, pl.ds(i * BN, BN)]
        cnt = jnp.maximum(jnp.sum(cp, axis=0), 1.0)[:, None]
        a = aggp[...]
        mean = (a[0] + a[1]) / cnt
        r = (jnp.dot(mean, wl[...], preferred_element_type=jnp.float32)
             + jnp.dot(x[...], wr[...], preferred_element_type=jnp.float32)
             + bl[...])
        o[...] = jnp.maximum(r, 0.0) if relu else r
    return body


def _dense(aggp, cntp, x, Wl, bl, Wr, relu):
    return pl.pallas_call(
        _dense_body(relu),
        grid=(pl.cdiv(N, BN),),
        in_specs=[
            pl.BlockSpec((2, BN, D), lambda i: (0, i, 0)),
            pl.BlockSpec((NW, 1, NPAD2), lambda i: (0, 0, 0)),
            pl.BlockSpec((BN, D), lambda i: (i, 0)),
            pl.BlockSpec((D, D), lambda i: (0, 0)),
            pl.BlockSpec((1, D), lambda i: (0, 0)),
            pl.BlockSpec((D, D), lambda i: (0, 0)),
        ],
        out_specs=pl.BlockSpec((BN, D), lambda i: (i, 0)),
        out_shape=jax.ShapeDtypeStruct((N, D), jnp.float32),
    )(aggp, cntp, x, Wl, bl.reshape(1, D), Wr)


def kernel(x, edge_index, W1l, b1l, W1r, W2l, b2l, W2r):
    npad = EPAD - E
    fill = jnp.arange(npad, dtype=jnp.int32) % 16
    src = jnp.concatenate([edge_index[0], fill])
    dst = jnp.concatenate([edge_index[1], N + fill])
    srcb = src.reshape(NBLKA, EBA)
    dstb = dst.reshape(NBLKA, EBA)
    (cnt,) = _sc_cnt(dst.reshape(NBLK, EB))
    (agg1,) = _sc_agg(x, srcb, dstb)
    h = _dense(agg1, cnt, x, W1l, b1l, W1r, relu=True)
    (agg2,) = _sc_agg(h, srcb, dstb)
    out = _dense(agg2, cnt, h, W2l, b2l, W2r, relu=False)
    return out


# speedup vs baseline: 1.4861x; 1.0309x over previous
import functools

import jax
import jax.numpy as jnp
from jax import lax
from jax.experimental import pallas as pl
from jax.experimental.pallas import tpu as pltpu
from jax.experimental.pallas import tpu_sc as plsc

N = 10000
E = 320000
D = 128
LANES = 16
NC = 2
NS = 16
NW = NC * NS
EB = 128
BPW = 80
NBLK = NW * BPW
EPAD = NBLK * EB
NPAD = N + 16
NPAD2 = 10240
RPS = 632
LAST_R = NPAD - 15 * RPS

_mesh = plsc.VectorSubcoreMesh(core_axis_name="c", subcore_axis_name="s")


def _zero_vmem(ref, nrows, ncols):
    zero16 = jnp.zeros((LANES,), jnp.float32)

    def zrow(i, c):
        for c8 in range(ncols // LANES):
            ref[i, pl.ds(c8 * LANES, LANES)] = zero16
        return c
    lax.fori_loop(0, nrows, zrow, 0)


def _zero_shared(dst_sh, src_buf, r0, nrows):
    nb = src_buf.shape[0]
    for k in range(nrows // nb):
        pltpu.sync_copy(src_buf, dst_sh.at[pl.ds(r0 + k * nb, nb)])
    tail = nrows % nb
    if tail:
        pltpu.sync_copy(src_buf.at[pl.ds(0, tail)],
                        dst_sh.at[pl.ds(r0 + (nrows // nb) * nb, tail)])


EBA = 128
NBLKA = EPAD // EBA
BPWA = NBLKA // NW
IDXCA = 8
NCH = BPWA // IDXCA

_agg_scratch = [
    pltpu.VMEM_SHARED((NPAD, D), jnp.float32),
    pltpu.VMEM((IDXCA, EBA), jnp.int32),
    pltpu.VMEM((IDXCA, EBA), jnp.int32),
    pltpu.VMEM((IDXCA, EBA), jnp.int32),
    pltpu.VMEM((IDXCA, EBA), jnp.int32),
    pltpu.VMEM((EBA, D), jnp.float32),
    pltpu.VMEM((EBA, D), jnp.float32),
    pltpu.SemaphoreType.DMA,
    pltpu.SemaphoreType.DMA,
    pltpu.SemaphoreType.DMA,
]
_agg_out_type = [jax.ShapeDtypeStruct((NC, NPAD, D), jnp.float32)]


def _sc_agg_body(feat, srcb, dstb, agg_out, agg_sh,
                 srcA, dstA, srcB, dstB, rows_a, rows_b, sga, sgb, sidx):
    cid = lax.axis_index("c")
    sid = lax.axis_index("s")
    w = cid * NS + sid
    r0 = sid * RPS

    _zero_vmem(rows_a, EBA, D)

    @pl.when(sid < NS - 1)
    def _():
        _zero_shared(agg_sh, rows_a, r0, RPS)

    @pl.when(sid == NS - 1)
    def _():
        _zero_shared(agg_sh, rows_a, r0, LAST_R)

    plsc.subcore_barrier()

    base = w * BPWA
    banks = ((srcA, dstA), (srcB, dstB))
    bufs = ((rows_a, sga), (rows_b, sgb))

    def load_chunk_start(c, bank):
        pltpu.async_copy(srcb.at[pl.ds(base + c * IDXCA, IDXCA)], bank[0],
                         sidx)
        pltpu.async_copy(dstb.at[pl.ds(base + c * IDXCA, IDXCA)], bank[1],
                         sidx)

    def load_chunk_wait(c, bank):
        pltpu.make_async_copy(srcb.at[pl.ds(base + c * IDXCA, IDXCA)],
                              bank[0], sidx).wait()
        pltpu.make_async_copy(dstb.at[pl.ds(base + c * IDXCA, IDXCA)],
                              bank[1], sidx).wait()

    load_chunk_start(0, banks[0])
    load_chunk_wait(0, banks[0])
    pltpu.async_copy(feat.at[banks[0][0].at[0]], rows_a, sga)
    pltpu.async_copy(feat.at[banks[0][0].at[1]], rows_b, sgb)

    for c in range(NCH):
        cur = banks[c % 2]
        nxt = banks[(c + 1) % 2]
        if c + 1 < NCH:
            load_chunk_start(c + 1, nxt)

        def blockstep(m2, t, prefetch):
            buf, sem = bufs[t]
            k = 2 * m2 + t
            pltpu.make_async_copy(feat.at[cur[0].at[k]], buf, sem).wait()
            pltpu.sync_copy(buf, agg_sh.at[cur[1].at[k]], add=True)
            if prefetch is not None:
                pbank, prow = prefetch
                pltpu.async_copy(feat.at[pbank[0].at[prow]], buf, sem)

        def pair(m2, carry):
            for t in (0, 1):
                blockstep(m2, t, (cur, 2 * m2 + t + 2))
            return carry
        lax.fori_loop(0, IDXCA // 2 - 1, pair, 0)

        if c + 1 < NCH:
            load_chunk_wait(c + 1, nxt)
        m_last = IDXCA // 2 - 1
        for t in (0, 1):
            pf = (nxt, t) if c + 1 < NCH else None
            blockstep(m_last, t, pf)

    plsc.subcore_barrier()

    @pl.when(sid < NS - 1)
    def _():
        pltpu.sync_copy(agg_sh.at[pl.ds(r0, RPS)],
                        agg_out.at[cid, pl.ds(r0, RPS)])

    @pl.when(sid == NS - 1)
    def _():
        pltpu.sync_copy(agg_sh.at[pl.ds(r0, LAST_R)],
                        agg_out.at[cid, pl.ds(r0, LAST_R)])


_sc_agg = pl.kernel(_sc_agg_body, mesh=_mesh, out_type=_agg_out_type,
                    scratch_types=_agg_scratch)


_cnt_out_type = [jax.ShapeDtypeStruct((NW, 1, NPAD2), jnp.float32)]
_cnt_scratch = [
    pltpu.VMEM((1, NPAD2), jnp.float32),
    pltpu.VMEM((BPW, EB), jnp.int32),
]


def _sc_cnt_body(dstb, cnt_out, hist, dst_v):
    cid = lax.axis_index("c")
    sid = lax.axis_index("s")
    w = cid * NS + sid

    zero16 = jnp.zeros((LANES,), jnp.float32)

    def zh(i, c):
        hist[0, pl.ds(i * LANES, LANES)] = zero16
        return c
    lax.fori_loop(0, NPAD2 // LANES, zh, 0)

    pltpu.sync_copy(dstb.at[pl.ds(w * BPW, BPW)], dst_v)

    zero16i = jnp.zeros((LANES,), jnp.int32)
    one16 = jnp.ones((LANES,), jnp.float32)

    def body(j, c):
        for k in range(EB // LANES):
            idx16 = dst_v[j, pl.ds(k * LANES, LANES)]
            plsc.addupdate_scatter(hist, [zero16i, idx16], one16)
        return c
    lax.fori_loop(0, BPW, body, 0)

    pltpu.sync_copy(hist, cnt_out.at[w])


_sc_cnt = pl.kernel(
    _sc_cnt_body, mesh=_mesh, out_type=_cnt_out_type,
    scratch_types=_cnt_scratch,
    compiler_params=pltpu.CompilerParams(needs_layout_passes=False))

BN = 2048


def _dense_body(relu):
    def body(aggp, cntp, x, wl, bl, wr, o):
        i = pl.program_id(0)
        cp = cntp[:, 0, pl.ds(i * BN, BN)]
        cnt = jnp.maximum(jnp.sum(cp, axis=0), 1.0)[:, None]
        a = aggp[...]
        mean = (a[0] + a[1]) / cnt
        r = (jnp.dot(mean, wl[...], preferred_element_type=jnp.float32)
             + jnp.dot(x[...], wr[...], preferred_element_type=jnp.float32)
             + bl[...])
        o[...] = jnp.maximum(r, 0.0) if relu else r
    return body


def _dense(aggp, cntp, x, Wl, bl, Wr, relu):
    return pl.pallas_call(
        _dense_body(relu),
        grid=(pl.cdiv(N, BN),),
        in_specs=[
            pl.BlockSpec((2, BN, D), lambda i: (0, i, 0)),
            pl.BlockSpec((NW, 1, NPAD2), lambda i: (0, 0, 0)),
            pl.BlockSpec((BN, D), lambda i: (i, 0)),
            pl.BlockSpec((D, D), lambda i: (0, 0)),
            pl.BlockSpec((1, D), lambda i: (0, 0)),
            pl.BlockSpec((D, D), lambda i: (0, 0)),
        ],
        out_specs=pl.BlockSpec((BN, D), lambda i: (i, 0)),
        out_shape=jax.ShapeDtypeStruct((N, D), jnp.float32),
    )(aggp, cntp, x, Wl, bl.reshape(1, D), Wr)


def kernel(x, edge_index, W1l, b1l, W1r, W2l, b2l, W2r):
    npad = EPAD - E
    fill = jnp.arange(npad, dtype=jnp.int32) % 16
    src = jnp.concatenate([edge_index[0], fill])
    dst = jnp.concatenate([edge_index[1], N + fill])
    srcb = src.reshape(NBLKA, EBA)
    dstb = dst.reshape(NBLKA, EBA)
    (cnt,) = _sc_cnt(dst.reshape(NBLK, EB))
    (agg1,) = _sc_agg(x, srcb, dstb)
    h = _dense(agg1, cnt, x, W1l, b1l, W1r, relu=True)
    (agg2,) = _sc_agg(h, srcb, dstb)
    out = _dense(agg2, cnt, h, W2l, b2l, W2r, relu=False)
    return out


# speedup vs baseline: 1.4865x; 1.0003x over previous
import jax
import jax.numpy as jnp
from jax import lax
from jax.experimental import pallas as pl
from jax.experimental.pallas import tpu as pltpu
from jax.experimental.pallas import tpu_sc as plsc

N = 10000
E = 320000
D = 128
LANES = 16
NC = 2
NS = 16
NW = NC * NS
EB = 128
BPW = 80
NBLK = NW * BPW
EPAD = NBLK * EB
NPAD = N + 16
NPAD2 = 10240
RPS = 632
LAST_R = NPAD - 15 * RPS

_mesh = plsc.VectorSubcoreMesh(core_axis_name="c", subcore_axis_name="s")


def _zero_vmem(ref, nrows, ncols):
    zero16 = jnp.zeros((LANES,), jnp.float32)

    def zrow(i, c):
        for c8 in range(ncols // LANES):
            ref[i, pl.ds(c8 * LANES, LANES)] = zero16
        return c
    lax.fori_loop(0, nrows, zrow, 0)


def _zero_shared(dst_sh, src_buf, r0, nrows):
    nb = src_buf.shape[0]
    for k in range(nrows // nb):
        pltpu.sync_copy(src_buf, dst_sh.at[pl.ds(r0 + k * nb, nb)])
    tail = nrows % nb
    if tail:
        pltpu.sync_copy(src_buf.at[pl.ds(0, tail)],
                        dst_sh.at[pl.ds(r0 + (nrows // nb) * nb, tail)])


EBA = 128
NBLKA = EPAD // EBA
BPWA = NBLKA // NW
IDXCA = 8
NCH = BPWA // IDXCA

_agg_scratch = [
    pltpu.VMEM_SHARED((NPAD, D), jnp.float32),
    pltpu.VMEM((IDXCA, EBA), jnp.int32),
    pltpu.VMEM((IDXCA, EBA), jnp.int32),
    pltpu.VMEM((IDXCA, EBA), jnp.int32),
    pltpu.VMEM((IDXCA, EBA), jnp.int32),
    pltpu.VMEM((EBA, D), jnp.float32),
    pltpu.VMEM((EBA, D), jnp.float32),
    pltpu.SemaphoreType.DMA,
    pltpu.SemaphoreType.DMA,
    pltpu.SemaphoreType.DMA,
]
_agg_out_type = [jax.ShapeDtypeStruct((NC, NPAD, D), jnp.float32)]


def _sc_agg_body(feat, srcb, dstb, agg_out, agg_sh,
                 srcA, dstA, srcB, dstB, rows_a, rows_b, sga, sgb, sidx):
    cid = lax.axis_index("c")
    sid = lax.axis_index("s")
    w = cid * NS + sid
    r0 = sid * RPS

    _zero_vmem(rows_a, EBA, D)

    @pl.when(sid < NS - 1)
    def _():
        _zero_shared(agg_sh, rows_a, r0, RPS)

    @pl.when(sid == NS - 1)
    def _():
        _zero_shared(agg_sh, rows_a, r0, LAST_R)

    plsc.subcore_barrier()

    base = w * BPWA
    banks = ((srcA, dstA), (srcB, dstB))
    bufs = ((rows_a, sga), (rows_b, sgb))

    def load_chunk_start(c, bank):
        pltpu.async_copy(srcb.at[pl.ds(base + c * IDXCA, IDXCA)], bank[0],
                         sidx)
        pltpu.async_copy(dstb.at[pl.ds(base + c * IDXCA, IDXCA)], bank[1],
                         sidx)

    def load_chunk_wait(c, bank):
        pltpu.make_async_copy(srcb.at[pl.ds(base + c * IDXCA, IDXCA)],
                              bank[0], sidx).wait()
        pltpu.make_async_copy(dstb.at[pl.ds(base + c * IDXCA, IDXCA)],
                              bank[1], sidx).wait()

    load_chunk_start(0, banks[0])
    load_chunk_wait(0, banks[0])
    pltpu.async_copy(feat.at[banks[0][0].at[0]], rows_a, sga)
    pltpu.async_copy(feat.at[banks[0][0].at[1]], rows_b, sgb)

    for c in range(NCH):
        cur = banks[c % 2]
        nxt = banks[(c + 1) % 2]
        if c + 1 < NCH:
            load_chunk_start(c + 1, nxt)

        def blockstep(m2, t, prefetch):
            buf, sem = bufs[t]
            k = 2 * m2 + t
            pltpu.make_async_copy(feat.at[cur[0].at[k]], buf, sem).wait()
            pltpu.sync_copy(buf, agg_sh.at[cur[1].at[k]], add=True)
            if prefetch is not None:
                pbank, prow = prefetch
                pltpu.async_copy(feat.at[pbank[0].at[prow]], buf, sem)

        def pair(m2, carry):
            for t in (0, 1):
                blockstep(m2, t, (cur, 2 * m2 + t + 2))
            return carry
        lax.fori_loop(0, IDXCA // 2 - 1, pair, 0)

        if c + 1 < NCH:
            load_chunk_wait(c + 1, nxt)
        m_last = IDXCA // 2 - 1
        for t in (0, 1):
            pf = (nxt, t) if c + 1 < NCH else None
            blockstep(m_last, t, pf)

    plsc.subcore_barrier()

    @pl.when(sid < NS - 1)
    def _():
        pltpu.sync_copy(agg_sh.at[pl.ds(r0, RPS)],
                        agg_out.at[cid, pl.ds(r0, RPS)])

    @pl.when(sid == NS - 1)
    def _():
        pltpu.sync_copy(agg_sh.at[pl.ds(r0, LAST_R)],
                        agg_out.at[cid, pl.ds(r0, LAST_R)])


_sc_agg = pl.kernel(_sc_agg_body, mesh=_mesh, out_type=_agg_out_type,
                    scratch_types=_agg_scratch)


_cnt_out_type = [jax.ShapeDtypeStruct((NW, 1, NPAD2), jnp.float32)]
_cnt_scratch = [
    pltpu.VMEM((1, NPAD2), jnp.float32),
    pltpu.VMEM((BPW, EB), jnp.int32),
]


def _sc_cnt_body(dstb, cnt_out, hist, dst_v):
    cid = lax.axis_index("c")
    sid = lax.axis_index("s")
    w = cid * NS + sid

    zero16 = jnp.zeros((LANES,), jnp.float32)

    def zh(i, c):
        hist[0, pl.ds(i * LANES, LANES)] = zero16
        return c
    lax.fori_loop(0, NPAD2 // LANES, zh, 0)

    pltpu.sync_copy(dstb.at[pl.ds(w * BPW, BPW)], dst_v)

    zero16i = jnp.zeros((LANES,), jnp.int32)
    one16 = jnp.ones((LANES,), jnp.float32)

    def body(j, c):
        for k in range(EB // LANES):
            idx16 = dst_v[j, pl.ds(k * LANES, LANES)]
            plsc.addupdate_scatter(hist, [zero16i, idx16], one16)
        return c
    lax.fori_loop(0, BPW, body, 0)

    pltpu.sync_copy(hist, cnt_out.at[w])


_sc_cnt = pl.kernel(
    _sc_cnt_body, mesh=_mesh, out_type=_cnt_out_type,
    scratch_types=_cnt_scratch,
    compiler_params=pltpu.CompilerParams(needs_layout_passes=False))

BN = 2048


def _dense_body(relu):
    def body(aggp, cntp, x, wl, bl, wr, o):
        i = pl.program_id(0)
        cp = cntp[:, 0, pl.ds(i * BN, BN)]
        cnt = jnp.maximum(jnp.sum(cp, axis=0), 1.0)[:, None]
        a = aggp[...]
        mean = (a[0] + a[1]) / cnt
        r = (jnp.dot(mean, wl[...], preferred_element_type=jnp.float32)
             + jnp.dot(x[...], wr[...], preferred_element_type=jnp.float32)
             + bl[...])
        o[...] = jnp.maximum(r, 0.0) if relu else r
    return body


def _dense(aggp, cntp, x, Wl, bl, Wr, relu):
    return pl.pallas_call(
        _dense_body(relu),
        grid=(pl.cdiv(N, BN),),
        in_specs=[
            pl.BlockSpec((2, BN, D), lambda i: (0, i, 0)),
            pl.BlockSpec((NW, 1, NPAD2), lambda i: (0, 0, 0)),
            pl.BlockSpec((BN, D), lambda i: (i, 0)),
            pl.BlockSpec((D, D), lambda i: (0, 0)),
            pl.BlockSpec((1, D), lambda i: (0, 0)),
            pl.BlockSpec((D, D), lambda i: (0, 0)),
        ],
        out_specs=pl.BlockSpec((BN, D), lambda i: (i, 0)),
        out_shape=jax.ShapeDtypeStruct((N, D), jnp.float32),
    )(aggp, cntp, x, Wl, bl.reshape(1, D), Wr)


def kernel(x, edge_index, W1l, b1l, W1r, W2l, b2l, W2r):
    npad = EPAD - E
    fill = jnp.arange(npad, dtype=jnp.int32) % 16
    src = jnp.concatenate([edge_index[0], fill])
    dst = jnp.concatenate([edge_index[1], N + fill])
    srcb = src.reshape(NBLKA, EBA)
    dstb = dst.reshape(NBLKA, EBA)
    (cnt,) = _sc_cnt(dst.reshape(NBLK, EB))
    (agg1,) = _sc_agg(x, srcb, dstb)
    h = _dense(agg1, cnt, x, W1l, b1l, W1r, relu=True)
    (agg2,) = _sc_agg(h, srcb, dstb)
    out = _dense(agg2, cnt, h, W2l, b2l, W2r, relu=False)
    return out
